# Initial kernel scaffold; baseline (speedup 1.0000x reference)
import jax
import jax.numpy as jnp
from jax.experimental import pallas as pl


def kernel(x, edge_index, W1, b1, W2, att_src2, att_dst2, b2, W3, b3, W4, att_src4, att_dst4, b4):
    raise NotImplementedError("write your pallas kernel here")


# speedup vs baseline: 1.0742x; 1.0742x over previous
import functools

import jax
import jax.numpy as jnp
from jax import lax
from jax.experimental import pallas as pl

_BM = 2000


def _mm_body(x_ref, w_ref, o_ref):
    o_ref[...] = jnp.dot(x_ref[...], w_ref[...],
                         preferred_element_type=jnp.float32)


def _matmul(x, w):
    m, k = x.shape
    n = w.shape[1]
    return pl.pallas_call(
        _mm_body,
        grid=(m // _BM,),
        in_specs=[pl.BlockSpec((_BM, k), lambda i: (i, 0)),
                  pl.BlockSpec((k, n), lambda i: (0, 0))],
        out_specs=pl.BlockSpec((_BM, n), lambda i: (i, 0)),
        out_shape=jax.ShapeDtypeStruct((m, n), jnp.float32),
    )(x, w)


def _bias_act_body(h_ref, b_ref, o_ref, *, relu):
    h = h_ref[...] + b_ref[...]
    o_ref[...] = jnp.maximum(h, 0.0) if relu else h


def _bias_act(h, b, relu):
    m, n = h.shape
    return pl.pallas_call(
        functools.partial(_bias_act_body, relu=relu),
        grid=(m // _BM,),
        in_specs=[pl.BlockSpec((_BM, n), lambda i: (i, 0)),
                  pl.BlockSpec((1, n), lambda i: (0, 0))],
        out_specs=pl.BlockSpec((_BM, n), lambda i: (i, 0)),
        out_shape=jax.ShapeDtypeStruct((m, n), jnp.float32),
    )(h, b.reshape(1, n))


def _lsm_body(h_ref, o_ref):
    h = h_ref[...]
    m = jnp.max(h, axis=1, keepdims=True)
    z = jnp.sum(jnp.exp(h - m), axis=1, keepdims=True)
    o_ref[...] = h - (m + jnp.log(z))


def _log_softmax(h):
    m, n = h.shape
    return pl.pallas_call(
        _lsm_body,
        grid=(m // _BM,),
        in_specs=[pl.BlockSpec((_BM, n), lambda i: (i, 0))],
        out_specs=pl.BlockSpec((_BM, n), lambda i: (i, 0)),
        out_shape=jax.ShapeDtypeStruct((m, n), jnp.float32),
    )(h)


def _seg_sum_rows(vals, seg, n):
    return jax.ops.segment_sum(vals, seg, num_segments=n,
                               indices_are_sorted=True)


def kernel(x, edge_index, W1, b1, W2, att_src2, att_dst2, b2,
           W3, b3, W4, att_src4, att_dst4, b4):
    n = x.shape[0]
    loop = jnp.arange(n, dtype=edge_index.dtype)
    src = jnp.concatenate([edge_index[0], loop])
    dst = jnp.concatenate([edge_index[1], loop])
    dst_s, src_s = lax.sort([dst, src], num_keys=1)

    deg = jax.ops.segment_sum(jnp.ones_like(dst_s, jnp.float32), dst_s,
                              num_segments=n, indices_are_sorted=True)
    dis = deg ** -0.5
    norm = dis[src_s] * dis[dst_s]

    xw = _matmul(x, W1)
    h = _seg_sum_rows(norm[:, None] * xw[src_s], dst_s, n)
    h = _bias_act(h, b1, relu=True)

    W2a = jnp.concatenate([W2, (W2 @ att_src2)[:, None],
                           (W2 @ att_dst2)[:, None]], axis=1)
    z = _matmul(h, W2a)
    xw, a_src, a_dst = z[:, :64], z[:, 64], z[:, 65]
    shift = jax.nn.leaky_relu(jnp.max(a_src) + a_dst, 0.2)
    e = jnp.exp(jax.nn.leaky_relu(a_src[src_s] + a_dst[dst_s], 0.2)
                - shift[dst_s])
    s = jax.ops.segment_sum(e, dst_s, num_segments=n,
                            indices_are_sorted=True)
    w = e / s[dst_s]
    h = _seg_sum_rows(w[:, None] * xw[src_s], dst_s, n)
    h = _bias_act(h, b2, relu=True)

    xw = _matmul(h, W3)
    h = _seg_sum_rows(norm[:, None] * xw[src_s], dst_s, n)
    h = _bias_act(h, b3, relu=True)

    W4a = jnp.concatenate([W4, (W4 @ att_src4)[:, None],
                           (W4 @ att_dst4)[:, None]], axis=1)
    z = _matmul(h, W4a)
    xw, a_src, a_dst = z[:, :2], z[:, 2], z[:, 3]
    shift = jax.nn.leaky_relu(jnp.max(a_src) + a_dst, 0.2)
    e = jnp.exp(jax.nn.leaky_relu(a_src[src_s] + a_dst[dst_s], 0.2)
                - shift[dst_s])
    s = jax.ops.segment_sum(e, dst_s, num_segments=n,
                            indices_are_sorted=True)
    w = e / s[dst_s]
    h = _seg_sum_rows(w[:, None] * xw[src_s], dst_s, n)
    h = _bias_act(h, b4, relu=False)

    return _log_softmax(h)


# speedup vs baseline: 20.2756x; 18.8757x over previous
import functools

import jax
import jax.numpy as jnp
from jax import lax
from jax.experimental import pallas as pl
from jax.experimental.pallas import tpu as pltpu
from jax.experimental.pallas import tpu_sc as plsc

_BM = 2000
_NC, _NS = 2, 16
_B = 128


def _mm_body(x_ref, w_ref, o_ref):
    o_ref[...] = jnp.dot(x_ref[...], w_ref[...],
                         preferred_element_type=jnp.float32)


def _matmul(x, w):
    m, k = x.shape
    n = w.shape[1]
    return pl.pallas_call(
        _mm_body,
        grid=(m // _BM,),
        in_specs=[pl.BlockSpec((_BM, k), lambda i: (i, 0)),
                  pl.BlockSpec((k, n), lambda i: (0, 0))],
        out_specs=pl.BlockSpec((_BM, n), lambda i: (i, 0)),
        out_shape=jax.ShapeDtypeStruct((m, n), jnp.float32),
    )(x, w)


def _post_body(h_ref, s_ref, b_ref, o_ref, *, relu, scale):
    h = h_ref[...]
    if scale:
        h = h * s_ref[...]
    h = h + b_ref[...]
    o_ref[...] = jnp.maximum(h, 0.0) if relu else h


def _post(h, scale_col, b, relu):
    m, n = h.shape
    use_scale = scale_col is not None
    if not use_scale:
        scale_col = jnp.zeros((m, 1), jnp.float32)
    return pl.pallas_call(
        functools.partial(_post_body, relu=relu, scale=use_scale),
        grid=(m // _BM,),
        in_specs=[pl.BlockSpec((_BM, n), lambda i: (i, 0)),
                  pl.BlockSpec((_BM, 1), lambda i: (i, 0)),
                  pl.BlockSpec((1, n), lambda i: (0, 0))],
        out_specs=pl.BlockSpec((_BM, n), lambda i: (i, 0)),
        out_shape=jax.ShapeDtypeStruct((m, n), jnp.float32),
    )(h, scale_col, b.reshape(1, n))


def _gat_pre_body(x_ref, w_ref, tab_ref, adst_ref, asrc_ref, *, d, dp):
    bm = x_ref.shape[0]
    z = jnp.dot(x_ref[...], w_ref[...], preferred_element_type=jnp.float32)
    ones = jnp.ones((bm, 1), jnp.float32)
    zeros = jnp.zeros((bm, dp - d - 1), jnp.float32)
    tab_ref[...] = jnp.concatenate([z[:, :d], ones, zeros], axis=1)
    adst_ref[...] = z[:, d + 1:d + 2]
    asrc_ref[...] = z[:, d:d + 1]


def _gat_pre(x, w_aug, dp):
    m, k = x.shape
    d = w_aug.shape[1] - 2
    return pl.pallas_call(
        functools.partial(_gat_pre_body, d=d, dp=dp),
        grid=(m // _BM,),
        in_specs=[pl.BlockSpec((_BM, k), lambda i: (i, 0)),
                  pl.BlockSpec((k, d + 2), lambda i: (0, 0))],
        out_specs=[pl.BlockSpec((_BM, dp), lambda i: (i, 0)),
                   pl.BlockSpec((_BM, 1), lambda i: (i, 0)),
                   pl.BlockSpec((_BM, 1), lambda i: (i, 0))],
        out_shape=[jax.ShapeDtypeStruct((m, dp), jnp.float32),
                   jax.ShapeDtypeStruct((m, 1), jnp.float32),
                   jax.ShapeDtypeStruct((m, 1), jnp.float32)],
    )(x, w_aug)


def _gat_post_body(a_ref, b_ref, o_ref, *, d, relu):
    a = a_ref[...]
    out = a[:, :d] / a[:, d:d + 1] + b_ref[...]
    o_ref[...] = jnp.maximum(out, 0.0) if relu else out


def _gat_post(acc, b, d, relu):
    m, dp = acc.shape
    return pl.pallas_call(
        functools.partial(_gat_post_body, d=d, relu=relu),
        grid=(m // _BM,),
        in_specs=[pl.BlockSpec((_BM, dp), lambda i: (i, 0)),
                  pl.BlockSpec((1, d), lambda i: (0, 0))],
        out_specs=pl.BlockSpec((_BM, d), lambda i: (i, 0)),
        out_shape=jax.ShapeDtypeStruct((m, d), jnp.float32),
    )(acc, b.reshape(1, d))


def _lsm_body(h_ref, o_ref):
    h = h_ref[...]
    m = jnp.max(h, axis=1, keepdims=True)
    z = jnp.sum(jnp.exp(h - m), axis=1, keepdims=True)
    o_ref[...] = h - (m + jnp.log(z))


def _log_softmax(h):
    m, n = h.shape
    return pl.pallas_call(
        _lsm_body,
        grid=(m // _BM,),
        in_specs=[pl.BlockSpec((_BM, n), lambda i: (i, 0))],
        out_specs=pl.BlockSpec((_BM, n), lambda i: (i, 0)),
        out_shape=jax.ShapeDtypeStruct((m, n), jnp.float32),
    )(h)


def _seg_sum_sc(tab, srcp, dstp, off, n_chunks, chunk, e_pad,
                adst=None, asrc=None, a16=None):
    dp = tab.shape[1]
    gat = adst is not None
    rows_per_tile = chunk // _NS
    slots = n_chunks // _NC
    zeros = jnp.zeros((chunk, dp), jnp.float32)
    mesh = plsc.VectorSubcoreMesh(core_axis_name="c", subcore_axis_name="s",
                                  num_cores=_NC, num_subcores=_NS)

    scratch = [
        pltpu.VMEM_SHARED((chunk + 8, dp), jnp.float32),
        pltpu.VMEM((16,), jnp.int32),
        pltpu.VMEM((_B,), jnp.int32),
        pltpu.VMEM((_B,), jnp.int32),
        pltpu.VMEM((_B,), jnp.int32),
        pltpu.VMEM((_B, dp), jnp.float32),
        pltpu.SemaphoreType.DMA,
    ]
    if gat:
        scratch += [pltpu.VMEM((_B,), jnp.float32),
                    pltpu.VMEM((_B,), jnp.float32),
                    pltpu.VMEM((16,), jnp.float32),
                    pltpu.SemaphoreType.DMA,
                    pltpu.SemaphoreType.DMA]

    @functools.partial(
        pl.kernel, mesh=mesh,
        compiler_params=pltpu.CompilerParams(use_tc_tiling_on_sc=False),
        out_type=jax.ShapeDtypeStruct((n_chunks * chunk, dp), jnp.float32),
        scratch_types=scratch,
    )
    def k(*refs):
        if gat:
            (tab_h, src_h, dst_h, off_h, z_h, adst_h, asrc_h, a_h, out_h,
             acc, off_v, src_v, dst_v, loc_v, rows_v, sem,
             asrc_v, adst_v, a_v, sem2, sem3) = refs
        else:
            (tab_h, src_h, dst_h, off_h, z_h, out_h,
             acc, off_v, src_v, dst_v, loc_v, rows_v, sem) = refs
        cid = lax.axis_index("c")
        sid = lax.axis_index("s")
        pltpu.sync_copy(off_h, off_v)
        if gat:
            pltpu.sync_copy(a_h, a_v)
        r0 = sid * rows_per_tile
        for j in range(slots):
            base = (2 * j + cid) * chunk
            pltpu.sync_copy(z_h.at[pl.ds(r0, rows_per_tile)],
                            acc.at[pl.ds(r0, rows_per_tile)])
            @pl.when(sid == 0)
            def _():
                pltpu.sync_copy(z_h.at[pl.ds(0, 8)],
                                acc.at[pl.ds(chunk, 8)])
            ov = off_v[pl.ds(0, 16)]
            e_lo = jnp.where(cid == 0, ov[2 * j], ov[2 * j + 1])
            e_hi = jnp.where(cid == 0, ov[2 * j + 1], ov[2 * j + 2])
            ln = e_hi - e_lo
            t0 = e_lo + (sid * ln) // _NS
            t1 = e_lo + ((sid + 1) * ln) // _NS
            a0 = (t0 // 8) * 8
            nb = (t1 - a0 + _B - 1) // _B
            plsc.subcore_barrier()

            def batch(b, carry):
                e0 = a0 + b * _B
                pltpu.sync_copy(src_h.at[pl.ds(e0, _B)], src_v)
                pltpu.sync_copy(dst_h.at[pl.ds(e0, _B)], dst_v)
                cp1 = pltpu.async_copy(tab_h.at[src_v], rows_v, sem)
                if gat:
                    cp2 = pltpu.async_copy(asrc_h.at[src_v], asrc_v, sem2)
                    cp3 = pltpu.async_copy(adst_h.at[dst_v], adst_v, sem3)
                    cp2.wait()
                    cp3.wait()
                cp1.wait()
                es = []
                for g in range(_B // 16):
                    pos = e0 + g * 16 + lax.iota(jnp.int32, 16)
                    d16 = dst_v[pl.ds(g * 16, 16)]
                    valid = (pos >= t0) & (pos < t1)
                    dloc = jnp.where(valid, d16 - base, chunk)
                    loc_v[pl.ds(g * 16, 16)] = dloc
                    if gat:
                        asrc = asrc_v[pl.ds(g * 16, 16)]
                        ad = adst_v[pl.ds(g * 16, 16)]
                        am = a_v[pl.ds(0, 16)]
                        alpha = asrc + ad
                        alpha = jnp.maximum(alpha, 0.2 * alpha)
                        sh = am + ad
                        sh = jnp.maximum(sh, 0.2 * sh)
                        es.append(jnp.exp(alpha - sh))
                if gat:
                    for g in range(_B // 16):
                        for l in range(16):
                            e16 = jnp.full((16,), es[g][l], jnp.float32)
                            r = g * 16 + l
                            for kk in range(dp // 16):
                                rows_v[r, pl.ds(kk * 16, 16)] = (
                                    rows_v[r, pl.ds(kk * 16, 16)] * e16)
                pltpu.sync_copy(rows_v, acc.at[loc_v], add=True)
                return carry

            lax.fori_loop(0, nb, batch, 0)
            plsc.subcore_barrier()
            pltpu.sync_copy(acc.at[pl.ds(r0, rows_per_tile)],
                            out_h.at[pl.ds(base + r0, rows_per_tile)])

    if gat:
        return k(tab, srcp, dstp, off, zeros, adst, asrc, a16)
    return k(tab, srcp, dstp, off, zeros)


def _chunk_offsets(dst_s, n_chunks, chunk):
    off_pad = 16
    bounds = jnp.arange(off_pad, dtype=jnp.int32) * chunk
    bounds = jnp.minimum(bounds, jnp.int32(n_chunks * chunk))
    return jnp.searchsorted(dst_s, bounds).astype(jnp.int32)


def kernel(x, edge_index, W1, b1, W2, att_src2, att_dst2, b2,
           W3, b3, W4, att_src4, att_dst4, b4):
    n = x.shape[0]
    loop = jnp.arange(n, dtype=edge_index.dtype)
    src = jnp.concatenate([edge_index[0], loop])
    dst = jnp.concatenate([edge_index[1], loop])
    dst_s, src_s = lax.sort([dst, src], num_keys=1)
    e_tot = dst_s.shape[0]
    e_pad = ((e_tot + 2 * _B + 7) // 8) * 8
    pad = e_pad - e_tot
    srcp = jnp.concatenate([src_s, jnp.zeros((pad,), src_s.dtype)])
    dstp = jnp.concatenate([dst_s, jnp.zeros((pad,), dst_s.dtype)])

    deg = jax.ops.segment_sum(jnp.ones_like(dst_s, jnp.float32), dst_s,
                              num_segments=n, indices_are_sorted=True)
    dis = deg ** -0.5
    dis_col = dis[:, None]

    c1, nch1 = 32768, 4
    off1 = _chunk_offsets(dst_s, nch1, c1)
    y = _matmul(dis_col * x, W1)
    acc = _seg_sum_sc(y, srcp, dstp, off1, nch1, c1, e_pad)
    h = _post(acc[:n], dis_col, b1, relu=True)

    c2, nch2 = 12800, 8
    off2 = _chunk_offsets(dst_s, nch2, c2)
    W2a = jnp.concatenate([W2, (W2 @ att_src2)[:, None],
                           (W2 @ att_dst2)[:, None]], axis=1)
    tab2, adst2, asrc2 = _gat_pre(h, W2a, dp=80)
    a16 = jnp.full((16,), jnp.max(asrc2), jnp.float32)
    acc = _seg_sum_sc(tab2, srcp, dstp, off2, nch2, c2, e_pad,
                      adst=adst2[:, 0], asrc=asrc2[:, 0], a16=a16)
    h = _gat_post(acc[:n], b2, d=64, relu=True)

    c3, nch3 = 8192, 14
    off3 = _chunk_offsets(dst_s, nch3, c3)
    y = _matmul(dis_col * h, W3)
    acc = _seg_sum_sc(y, srcp, dstp, off3, nch3, c3, e_pad)
    h = _post(acc[:n], dis_col, b3, relu=True)

    c4, nch4 = 50000, 2
    off4 = _chunk_offsets(dst_s, nch4, c4)
    W4a = jnp.concatenate([W4, (W4 @ att_src4)[:, None],
                           (W4 @ att_dst4)[:, None]], axis=1)
    tab4, adst4, asrc4 = _gat_pre(h, W4a, dp=16)
    a16 = jnp.full((16,), jnp.max(asrc4), jnp.float32)
    acc = _seg_sum_sc(tab4, srcp, dstp, off4, nch4, c4, e_pad,
                      adst=adst4[:, 0], asrc=asrc4[:, 0], a16=a16)
    h = _gat_post(acc[:n], b4, d=2, relu=False)

    return _log_softmax(h)


# speedup vs baseline: 26.5730x; 1.3106x over previous
import functools

import jax
import jax.numpy as jnp
from jax import lax
from jax.experimental import pallas as pl
from jax.experimental.pallas import tpu as pltpu
from jax.experimental.pallas import tpu_sc as plsc

_BM = 2000
_NC, _NS = 2, 16
_B = 128


def _mm_body(x_ref, w_ref, o_ref):
    o_ref[...] = jnp.dot(x_ref[...], w_ref[...],
                         preferred_element_type=jnp.float32)


def _matmul(x, w):
    m, k = x.shape
    n = w.shape[1]
    return pl.pallas_call(
        _mm_body,
        grid=(m // _BM,),
        in_specs=[pl.BlockSpec((_BM, k), lambda i: (i, 0)),
                  pl.BlockSpec((k, n), lambda i: (0, 0))],
        out_specs=pl.BlockSpec((_BM, n), lambda i: (i, 0)),
        out_shape=jax.ShapeDtypeStruct((m, n), jnp.float32),
    )(x, w)


def _post_body(h_ref, s_ref, b_ref, o_ref, *, relu, scale):
    h = h_ref[...]
    if scale:
        h = h * s_ref[...]
    h = h + b_ref[...]
    o_ref[...] = jnp.maximum(h, 0.0) if relu else h


def _post(h, scale_col, b, relu):
    m, n = h.shape
    use_scale = scale_col is not None
    if not use_scale:
        scale_col = jnp.zeros((m, 1), jnp.float32)
    return pl.pallas_call(
        functools.partial(_post_body, relu=relu, scale=use_scale),
        grid=(m // _BM,),
        in_specs=[pl.BlockSpec((_BM, n), lambda i: (i, 0)),
                  pl.BlockSpec((_BM, 1), lambda i: (i, 0)),
                  pl.BlockSpec((1, n), lambda i: (0, 0))],
        out_specs=pl.BlockSpec((_BM, n), lambda i: (i, 0)),
        out_shape=jax.ShapeDtypeStruct((m, n), jnp.float32),
    )(h, scale_col, b.reshape(1, n))


def _gat_pre_body(x_ref, w_ref, tab_ref, adst_ref, asrc_ref, *, d, dp):
    bm = x_ref.shape[0]
    z = jnp.dot(x_ref[...], w_ref[...], preferred_element_type=jnp.float32)
    ones = jnp.ones((bm, 1), jnp.float32)
    zeros = jnp.zeros((bm, dp - d - 1), jnp.float32)
    tab_ref[...] = jnp.concatenate([z[:, :d], ones, zeros], axis=1)
    adst_ref[...] = z[:, d + 1:d + 2]
    asrc_ref[...] = z[:, d:d + 1]


def _gat_pre(x, w_aug, dp):
    m, k = x.shape
    d = w_aug.shape[1] - 2
    return pl.pallas_call(
        functools.partial(_gat_pre_body, d=d, dp=dp),
        grid=(m // _BM,),
        in_specs=[pl.BlockSpec((_BM, k), lambda i: (i, 0)),
                  pl.BlockSpec((k, d + 2), lambda i: (0, 0))],
        out_specs=[pl.BlockSpec((_BM, dp), lambda i: (i, 0)),
                   pl.BlockSpec((_BM, 1), lambda i: (i, 0)),
                   pl.BlockSpec((_BM, 1), lambda i: (i, 0))],
        out_shape=[jax.ShapeDtypeStruct((m, dp), jnp.float32),
                   jax.ShapeDtypeStruct((m, 1), jnp.float32),
                   jax.ShapeDtypeStruct((m, 1), jnp.float32)],
    )(x, w_aug)


def _gat_post_body(a_ref, b_ref, o_ref, *, d, relu):
    a = a_ref[...]
    out = a[:, :d] / a[:, d:d + 1] + b_ref[...]
    o_ref[...] = jnp.maximum(out, 0.0) if relu else out


def _gat_post(acc, b, d, relu):
    m, dp = acc.shape
    return pl.pallas_call(
        functools.partial(_gat_post_body, d=d, relu=relu),
        grid=(m // _BM,),
        in_specs=[pl.BlockSpec((_BM, dp), lambda i: (i, 0)),
                  pl.BlockSpec((1, d), lambda i: (0, 0))],
        out_specs=pl.BlockSpec((_BM, d), lambda i: (i, 0)),
        out_shape=jax.ShapeDtypeStruct((m, d), jnp.float32),
    )(acc, b.reshape(1, d))


def _lsm_body(h_ref, o_ref):
    h = h_ref[...]
    m = jnp.max(h, axis=1, keepdims=True)
    z = jnp.sum(jnp.exp(h - m), axis=1, keepdims=True)
    o_ref[...] = h - (m + jnp.log(z))


def _log_softmax(h):
    m, n = h.shape
    return pl.pallas_call(
        _lsm_body,
        grid=(m // _BM,),
        in_specs=[pl.BlockSpec((_BM, n), lambda i: (i, 0))],
        out_specs=pl.BlockSpec((_BM, n), lambda i: (i, 0)),
        out_shape=jax.ShapeDtypeStruct((m, n), jnp.float32),
    )(h)


def _seg_sum_sc(tab, srcp, dstp, off, n_chunks, chunk, e_pad,
                adst=None, asrc=None, a16=None):
    dp = tab.shape[1]
    gat = adst is not None
    rows_per_tile = chunk // _NS
    slots = n_chunks // _NC
    zeros = jnp.zeros((chunk, dp), jnp.float32)
    mesh = plsc.VectorSubcoreMesh(core_axis_name="c", subcore_axis_name="s",
                                  num_cores=_NC, num_subcores=_NS)

    sub = 2 if (gat and dp > 16) else 4
    scratch = [
        pltpu.VMEM_SHARED((chunk + 8, dp), jnp.float32),
        pltpu.VMEM((16,), jnp.int32),
        pltpu.VMEM((sub, _B), jnp.int32),
        pltpu.VMEM((sub, _B), jnp.int32),
        pltpu.VMEM((sub, _B), jnp.int32),
        pltpu.VMEM((sub, _B, dp), jnp.float32),
        pltpu.SemaphoreType.DMA,
        pltpu.SemaphoreType.DMA,
    ]
    if gat:
        scratch += [pltpu.VMEM((sub, _B), jnp.float32),
                    pltpu.VMEM((sub, _B), jnp.float32),
                    pltpu.VMEM((16,), jnp.float32),
                    pltpu.SemaphoreType.DMA,
                    pltpu.SemaphoreType.DMA]

    @functools.partial(
        pl.kernel, mesh=mesh,
        compiler_params=pltpu.CompilerParams(use_tc_tiling_on_sc=False),
        out_type=jax.ShapeDtypeStruct((n_chunks * chunk, dp), jnp.float32),
        scratch_types=scratch,
    )
    def k(*refs):
        if gat:
            (tab_h, src_h, dst_h, off_h, z_h, adst_h, asrc_h, a_h, out_h,
             acc, off_v, src_v, dst_v, loc_v, rows_v, sem, isem,
             asrc_v, adst_v, a_v, sem2, sem3) = refs
        else:
            (tab_h, src_h, dst_h, off_h, z_h, out_h,
             acc, off_v, src_v, dst_v, loc_v, rows_v, sem, isem) = refs
        cid = lax.axis_index("c")
        sid = lax.axis_index("s")
        pltpu.sync_copy(off_h, off_v)
        if gat:
            pltpu.sync_copy(a_h, a_v)
        r0 = sid * rows_per_tile
        for j in range(slots):
            base = (2 * j + cid) * chunk
            pltpu.sync_copy(z_h.at[pl.ds(r0, rows_per_tile)],
                            acc.at[pl.ds(r0, rows_per_tile)])
            @pl.when(sid == 0)
            def _():
                pltpu.sync_copy(z_h.at[pl.ds(0, 8)],
                                acc.at[pl.ds(chunk, 8)])
            ov = off_v[pl.ds(0, 16)]
            e_lo = jnp.where(cid == 0, ov[2 * j], ov[2 * j + 1])
            e_hi = jnp.where(cid == 0, ov[2 * j + 1], ov[2 * j + 2])
            ln = e_hi - e_lo
            t0 = e_lo + (sid * ln) // _NS
            t1 = e_lo + ((sid + 1) * ln) // _NS
            a0 = (t0 // 8) * 8
            step = sub * _B
            nb = (t1 - a0 + step - 1) // step
            plsc.subcore_barrier()

            def batch(b, carry):
                e0 = a0 + b * step
                cps = []
                for j in range(sub):
                    cps.append(pltpu.async_copy(
                        src_h.at[pl.ds(e0 + j * _B, _B)], src_v.at[j], isem))
                    cps.append(pltpu.async_copy(
                        dst_h.at[pl.ds(e0 + j * _B, _B)], dst_v.at[j], isem))
                for c in cps:
                    c.wait()
                gps = []
                for j in range(sub):
                    gps.append(pltpu.async_copy(
                        tab_h.at[src_v.at[j]], rows_v.at[j], sem))
                    if gat:
                        gps.append(pltpu.async_copy(
                            asrc_h.at[src_v.at[j]], asrc_v.at[j], sem2))
                        gps.append(pltpu.async_copy(
                            adst_h.at[dst_v.at[j]], adst_v.at[j], sem3))
                for c in gps:
                    c.wait()
                for j in range(sub):
                    es = []
                    for g in range(_B // 16):
                        pos = e0 + j * _B + g * 16 + lax.iota(jnp.int32, 16)
                        d16 = dst_v[j, pl.ds(g * 16, 16)]
                        valid = (pos >= t0) & (pos < t1)
                        dloc = jnp.where(valid, d16 - base, chunk)
                        loc_v[j, pl.ds(g * 16, 16)] = dloc
                        if gat:
                            asrc = asrc_v[j, pl.ds(g * 16, 16)]
                            ad = adst_v[j, pl.ds(g * 16, 16)]
                            am = a_v[pl.ds(0, 16)]
                            alpha = asrc + ad
                            alpha = jnp.maximum(alpha, 0.2 * alpha)
                            sh = am + ad
                            sh = jnp.maximum(sh, 0.2 * sh)
                            es.append(jnp.exp(alpha - sh))
                    if gat:
                        for g in range(_B // 16):
                            for l in range(16):
                                e16 = jnp.full((16,), es[g][l], jnp.float32)
                                r = g * 16 + l
                                for kk in range(dp // 16):
                                    rows_v[j, r, pl.ds(kk * 16, 16)] = (
                                        rows_v[j, r, pl.ds(kk * 16, 16)]
                                        * e16)
                for j in range(sub):
                    pltpu.sync_copy(rows_v.at[j], acc.at[loc_v.at[j]],
                                    add=True)
                return carry

            lax.fori_loop(0, nb, batch, 0)
            plsc.subcore_barrier()
            pltpu.sync_copy(acc.at[pl.ds(r0, rows_per_tile)],
                            out_h.at[pl.ds(base + r0, rows_per_tile)])

    if gat:
        return k(tab, srcp, dstp, off, zeros, adst, asrc, a16)
    return k(tab, srcp, dstp, off, zeros)


def _chunk_offsets(dst_s, n_chunks, chunk):
    off_pad = 16
    bounds = jnp.arange(off_pad, dtype=jnp.int32) * chunk
    bounds = jnp.minimum(bounds, jnp.int32(n_chunks * chunk))
    return jnp.searchsorted(dst_s, bounds).astype(jnp.int32)


def kernel(x, edge_index, W1, b1, W2, att_src2, att_dst2, b2,
           W3, b3, W4, att_src4, att_dst4, b4):
    n = x.shape[0]
    loop = jnp.arange(n, dtype=edge_index.dtype)
    src = jnp.concatenate([edge_index[0], loop])
    dst = jnp.concatenate([edge_index[1], loop])
    dst_s, src_s = lax.sort([dst, src], num_keys=1)
    e_tot = dst_s.shape[0]
    e_pad = ((e_tot + 8 * _B + 7) // 8) * 8
    pad = e_pad - e_tot
    srcp = jnp.concatenate([src_s, jnp.zeros((pad,), src_s.dtype)])
    dstp = jnp.concatenate([dst_s, jnp.zeros((pad,), dst_s.dtype)])

    deg = jax.ops.segment_sum(jnp.ones_like(dst_s, jnp.float32), dst_s,
                              num_segments=n, indices_are_sorted=True)
    dis = deg ** -0.5
    dis_col = dis[:, None]

    c1, nch1 = 32768, 4
    off1 = _chunk_offsets(dst_s, nch1, c1)
    y = _matmul(dis_col * x, W1)
    acc = _seg_sum_sc(y, srcp, dstp, off1, nch1, c1, e_pad)
    h = _post(acc[:n], dis_col, b1, relu=True)

    c2, nch2 = 12800, 8
    off2 = _chunk_offsets(dst_s, nch2, c2)
    W2a = jnp.concatenate([W2, (W2 @ att_src2)[:, None],
                           (W2 @ att_dst2)[:, None]], axis=1)
    tab2, adst2, asrc2 = _gat_pre(h, W2a, dp=80)
    a16 = jnp.full((16,), jnp.max(asrc2), jnp.float32)
    acc = _seg_sum_sc(tab2, srcp, dstp, off2, nch2, c2, e_pad,
                      adst=adst2[:, 0], asrc=asrc2[:, 0], a16=a16)
    h = _gat_post(acc[:n], b2, d=64, relu=True)

    c3, nch3 = 7168, 14
    off3 = _chunk_offsets(dst_s, nch3, c3)
    y = _matmul(dis_col * h, W3)
    acc = _seg_sum_sc(y, srcp, dstp, off3, nch3, c3, e_pad)
    h = _post(acc[:n], dis_col, b3, relu=True)

    c4, nch4 = 50000, 2
    off4 = _chunk_offsets(dst_s, nch4, c4)
    W4a = jnp.concatenate([W4, (W4 @ att_src4)[:, None],
                           (W4 @ att_dst4)[:, None]], axis=1)
    tab4, adst4, asrc4 = _gat_pre(h, W4a, dp=16)
    a16 = jnp.full((16,), jnp.max(asrc4), jnp.float32)
    acc = _seg_sum_sc(tab4, srcp, dstp, off4, nch4, c4, e_pad,
                      adst=adst4[:, 0], asrc=asrc4[:, 0], a16=a16)
    h = _gat_post(acc[:n], b4, d=2, relu=False)

    return _log_softmax(h)
